# baseline (speedup 1.0000x reference)
import functools

import jax
import jax.numpy as jnp
from jax import lax
from jax.experimental import pallas as pl
from jax.experimental.pallas import tpu as pltpu
from jax.experimental.pallas import tpu_sc as plsc


def _table_body(ev_ref, hd_ref, es_ref, hs_ref, w_ref, b_ref, out_ref):
    ev = ev_ref[...] * es_ref[0]
    hd = hd_ref[...] * hs_ref[0]
    w = w_ref[...]
    e_dim = ev.shape[1]
    we = w[:, :e_dim]
    wh = w[:, e_dim:]
    t = lax.dot_general(ev, we, (((1,), (1,)), ((), ())),
                        preferred_element_type=jnp.float32)
    t += lax.dot_general(hd, wh, (((1,), (1,)), ((), ())),
                         preferred_element_type=jnp.float32)
    out_ref[...] = t + b_ref[...][None, :]


def _build_table(ev, hd, es, hs, w, b):
    n = ev.shape[0]
    d_out = w.shape[0]
    return pl.pallas_call(
        _table_body,
        out_shape=jax.ShapeDtypeStruct((n, d_out), jnp.float32),
        in_specs=[
            pl.BlockSpec(memory_space=pltpu.VMEM),
            pl.BlockSpec(memory_space=pltpu.VMEM),
            pl.BlockSpec(memory_space=pltpu.SMEM),
            pl.BlockSpec(memory_space=pltpu.SMEM),
            pl.BlockSpec(memory_space=pltpu.VMEM),
            pl.BlockSpec(memory_space=pltpu.VMEM),
        ],
        out_specs=pl.BlockSpec(memory_space=pltpu.VMEM),
    )(ev, hd, es, hs, w, b)


_NC = 2
_NS = 16
_NW = _NC * _NS
_CHUNK = 256
_NBUF = 2


_L = 16
_GRP = _CHUNK // _L
_D = 128


def _gather_body(b_per_w, n_chunk, table_hbm, idx_hbm, out_hbm,
                 tab_v, idx_v, *bufs_and_sems):
    wid = lax.axis_index("s") * _NC + lax.axis_index("c")
    base = wid * b_per_w
    pltpu.sync_copy(table_hbm, tab_v)
    pltpu.sync_copy(idx_hbm.at[pl.ds(base, b_per_w)], idx_v)

    bufs = bufs_and_sems[:_NBUF]
    osems = bufs_and_sems[_NBUF:]

    iota = lax.iota(jnp.int32, _L)
    cols = [iota + (j * _L) for j in range(_D // _L)]

    def fill_chunk(i, buf):
        @plsc.parallel_loop(0, _CHUNK, step=1, unroll=8)
        def _(p):
            lbl = plsc.load_gather(
                idx_v, [jnp.full((_L,), i * _CHUNK + p, jnp.int32)])
            flat_base = lbl * _D
            vs = [plsc.load_gather(tab_v, [flat_base + cols[j]])
                  for j in range(_D // _L)]
            for j in range(_D // _L):
                buf[pl.ds(p * _D + j * _L, _L)] = vs[j]

    def chunk_body(i, _):
        for s in range(_NBUF):
            @pl.when(lax.rem(i, _NBUF) == s)
            def _():
                @pl.when(i >= _NBUF)
                def _():
                    pltpu.make_async_copy(
                        bufs[s],
                        out_hbm.at[pl.ds((base + (i - _NBUF) * _CHUNK) * _D,
                                         _CHUNK * _D)],
                        osems[s]).wait()
                fill_chunk(i, bufs[s])
                pltpu.async_copy(
                    bufs[s],
                    out_hbm.at[pl.ds((base + i * _CHUNK) * _D, _CHUNK * _D)],
                    osems[s])
        return _

    lax.fori_loop(0, n_chunk, chunk_body, None)

    for i_last in range(n_chunk - _NBUF, n_chunk):
        s = i_last % _NBUF
        pltpu.make_async_copy(
            bufs[s],
            out_hbm.at[pl.ds((base + i_last * _CHUNK) * _D, _CHUNK * _D)],
            osems[s]).wait()


def _gather_rows(table_flat, flat_idx):
    b = flat_idx.shape[0]
    b_per_w = b // _NW
    n_chunk = b_per_w // _CHUNK
    mesh = plsc.VectorSubcoreMesh(core_axis_name="c", subcore_axis_name="s")
    run = pl.kernel(
        functools.partial(_gather_body, b_per_w, n_chunk),
        mesh=mesh,
        compiler_params=pltpu.CompilerParams(needs_layout_passes=False),
        out_type=jax.ShapeDtypeStruct((b * _D,), jnp.float32),
        scratch_types=(
            [pltpu.VMEM(table_flat.shape, jnp.float32),
             pltpu.VMEM((b_per_w,), jnp.int32)]
            + [pltpu.VMEM((_CHUNK * _D,), jnp.float32)] * _NBUF
            + [pltpu.SemaphoreType.DMA] * _NBUF
        ),
    )
    return run(table_flat, flat_idx)


def kernel(labels, elmes_vectors, hdc_vectors, elmes_scale, hdc_scale, W, b):
    batch, hist = labels.shape
    d_out = W.shape[0]
    table = _build_table(elmes_vectors, hdc_vectors, elmes_scale,
                         hdc_scale, W, b)
    flat = labels.reshape(-1).astype(jnp.int32)
    out = _gather_rows(table.reshape(-1), flat)
    return out.reshape(batch, hist, d_out)


# speedup vs baseline: 1.0597x; 1.0597x over previous
import functools

import jax
import jax.numpy as jnp
from jax import lax
from jax.experimental import pallas as pl
from jax.experimental.pallas import tpu as pltpu
from jax.experimental.pallas import tpu_sc as plsc


def _table_body(ev_ref, hd_ref, es_ref, hs_ref, w_ref, b_ref, out_ref):
    ev = ev_ref[...] * es_ref[0]
    hd = hd_ref[...] * hs_ref[0]
    w = w_ref[...]
    e_dim = ev.shape[1]
    we = w[:, :e_dim]
    wh = w[:, e_dim:]
    t = lax.dot_general(ev, we, (((1,), (1,)), ((), ())),
                        preferred_element_type=jnp.float32)
    t += lax.dot_general(hd, wh, (((1,), (1,)), ((), ())),
                         preferred_element_type=jnp.float32)
    out_ref[...] = t + b_ref[...][None, :]


def _build_table(ev, hd, es, hs, w, b):
    n = ev.shape[0]
    d_out = w.shape[0]
    return pl.pallas_call(
        _table_body,
        out_shape=jax.ShapeDtypeStruct((n, d_out), jnp.float32),
        in_specs=[
            pl.BlockSpec(memory_space=pltpu.VMEM),
            pl.BlockSpec(memory_space=pltpu.VMEM),
            pl.BlockSpec(memory_space=pltpu.SMEM),
            pl.BlockSpec(memory_space=pltpu.SMEM),
            pl.BlockSpec(memory_space=pltpu.VMEM),
            pl.BlockSpec(memory_space=pltpu.VMEM),
        ],
        out_specs=pl.BlockSpec(memory_space=pltpu.VMEM),
    )(ev, hd, es, hs, w, b)


_NC = 2
_NS = 16
_NW = _NC * _NS
_CHUNK = 256
_NBUF = 2


_L = 16
_GRP = _CHUNK // _L
_D = 128


def _gather_body(b_per_w, n_chunk, table_hbm, idx_hbm, out_hbm,
                 tab_v, idx_v, *bufs_and_sems):
    wid = lax.axis_index("s") * _NC + lax.axis_index("c")
    base = wid * b_per_w

    bufs = bufs_and_sems[:_NBUF]
    osems = bufs_and_sems[_NBUF:]

    pltpu.async_copy(table_hbm, tab_v, osems[0])
    pltpu.async_copy(idx_hbm.at[pl.ds(base, b_per_w)], idx_v, osems[1])
    pltpu.make_async_copy(table_hbm, tab_v, osems[0]).wait()
    pltpu.make_async_copy(idx_hbm.at[pl.ds(base, b_per_w)], idx_v,
                          osems[1]).wait()

    iota = lax.iota(jnp.int32, _L)
    cols = [iota + (j * _L) for j in range(_D // _L)]

    def fill_chunk(i, buf):
        @plsc.parallel_loop(0, _CHUNK, step=1, unroll=4)
        def _(p):
            lbl = plsc.load_gather(
                idx_v, [jnp.full((_L,), i * _CHUNK + p, jnp.int32)])
            flat_base = lbl * _D
            vs = [plsc.load_gather(tab_v, [flat_base + cols[j]])
                  for j in range(_D // _L)]
            for j in range(_D // _L):
                buf[pl.ds(p * _D + j * _L, _L)] = vs[j]

    def chunk_body(i, _):
        for s in range(_NBUF):
            @pl.when(lax.rem(i, _NBUF) == s)
            def _():
                @pl.when(i >= _NBUF)
                def _():
                    pltpu.make_async_copy(
                        bufs[s],
                        out_hbm.at[pl.ds((base + (i - _NBUF) * _CHUNK) * _D,
                                         _CHUNK * _D)],
                        osems[s]).wait()
                fill_chunk(i, bufs[s])
                pltpu.async_copy(
                    bufs[s],
                    out_hbm.at[pl.ds((base + i * _CHUNK) * _D, _CHUNK * _D)],
                    osems[s])
        return _

    lax.fori_loop(0, n_chunk, chunk_body, None)

    for i_last in range(n_chunk - _NBUF, n_chunk):
        s = i_last % _NBUF
        pltpu.make_async_copy(
            bufs[s],
            out_hbm.at[pl.ds((base + i_last * _CHUNK) * _D, _CHUNK * _D)],
            osems[s]).wait()


def _gather_rows(table_flat, flat_idx):
    b = flat_idx.shape[0]
    b_per_w = b // _NW
    n_chunk = b_per_w // _CHUNK
    mesh = plsc.VectorSubcoreMesh(core_axis_name="c", subcore_axis_name="s")
    run = pl.kernel(
        functools.partial(_gather_body, b_per_w, n_chunk),
        mesh=mesh,
        compiler_params=pltpu.CompilerParams(needs_layout_passes=False),
        out_type=jax.ShapeDtypeStruct((b * _D,), jnp.float32),
        scratch_types=(
            [pltpu.VMEM(table_flat.shape, jnp.float32),
             pltpu.VMEM((b_per_w,), jnp.int32)]
            + [pltpu.VMEM((_CHUNK * _D,), jnp.float32)] * _NBUF
            + [pltpu.SemaphoreType.DMA] * _NBUF
        ),
    )
    return run(table_flat, flat_idx)


def kernel(labels, elmes_vectors, hdc_vectors, elmes_scale, hdc_scale, W, b):
    batch, hist = labels.shape
    d_out = W.shape[0]
    table = _build_table(elmes_vectors, hdc_vectors, elmes_scale,
                         hdc_scale, W, b)
    flat = labels.reshape(-1).astype(jnp.int32)
    out = _gather_rows(table.reshape(-1), flat)
    return out.reshape(batch, hist, d_out)
